# baseline (speedup 1.0000x reference)
import functools

import jax
import jax.numpy as jnp
from jax import lax
from jax.experimental import pallas as pl
from jax.experimental.pallas import tpu as pltpu
from jax.experimental.pallas import tpu_sc as plsc

T = 2048
D = 2048
E = 8
F = 1408
TOPK = 2

BT = 256
B = 128
NB = (T * TOPK) // B + E
NR = NB * B
EP = 128

NW = 32
L = 16
TPW = T // NW
CHT = 16
NCH = TPW // CHT
CHC = 8
NCC = TPW // CHC


def _router_kernel(x_ref, gwt_ref, i0_ref, i1_ref, w0_ref, w1_ref,
                   p0_ref, p1_ref, cnt_ref, cnt_scr):
    b = pl.program_id(0)

    @pl.when(b == 0)
    def _():
        cnt_scr[...] = jnp.zeros_like(cnt_scr)

    x = x_ref[...]
    logits = jax.lax.dot_general(
        x, gwt_ref[...], (((1,), (0,)), ((), ())),
        preferred_element_type=jnp.float32)
    col = jax.lax.broadcasted_iota(jnp.int32, logits.shape, 1)
    neg = jnp.float32(-1e30)
    logits = jnp.where(col < E, logits, neg)
    m0 = jnp.max(logits, axis=1, keepdims=True)
    i0 = jnp.min(jnp.where(logits == m0, col, EP), axis=1, keepdims=True)
    l2 = jnp.where(col == i0, neg, logits)
    m1 = jnp.max(l2, axis=1, keepdims=True)
    i1 = jnp.min(jnp.where(l2 == m1, col, EP), axis=1, keepdims=True)
    w0 = jax.nn.sigmoid(m0 - m1)
    w1 = 1.0 - w0

    oh0 = (col == i0).astype(jnp.float32)
    oh1 = (col == i1).astype(jnp.float32)
    occ = oh0 + oh1
    r = jax.lax.broadcasted_iota(jnp.int32, (BT, BT), 0)
    c = jax.lax.broadcasted_iota(jnp.int32, (BT, BT), 1)
    ls = (c < r).astype(jnp.float32)
    pref = jax.lax.dot_general(
        ls, occ, (((1,), (0,)), ((), ())),
        preferred_element_type=jnp.float32)
    cnt = cnt_scr[0:1, :]
    base = pref + cnt
    p0 = jnp.sum(oh0 * base, axis=1, keepdims=True)
    p1 = jnp.sum(oh1 * base, axis=1, keepdims=True)
    cnt_new = cnt + jnp.sum(occ, axis=0, keepdims=True)
    cnt_scr[...] = jnp.broadcast_to(cnt_new, cnt_scr.shape)
    cnt_ref[...] = jnp.broadcast_to(cnt_new, cnt_ref.shape)

    i0_ref[...] = jnp.broadcast_to(i0, i0_ref.shape)
    i1_ref[...] = jnp.broadcast_to(i1, i1_ref.shape)
    w0_ref[...] = jnp.broadcast_to(w0, w0_ref.shape)
    w1_ref[...] = jnp.broadcast_to(w1, w1_ref.shape)
    p0_ref[...] = jnp.broadcast_to(p0, p0_ref.shape)
    p1_ref[...] = jnp.broadcast_to(p1, p1_ref.shape)


def _sc_dispatch(x_hbm, d0_hbm, d1_hbm, xg_hbm, i0_v, i1_v,
                 rows_a, rows_b, sem_a, sem_b):
    wid = lax.axis_index("s") * 2 + lax.axis_index("c")
    base = wid * TPW
    pltpu.sync_copy(d0_hbm.at[pl.ds(wid * NCH, NCH)], i0_v)
    pltpu.sync_copy(d1_hbm.at[pl.ds(wid * NCH, NCH)], i1_v)
    bufs = (rows_a, rows_b)
    sems = (sem_a, sem_b)
    copies = [None, None]

    def start(c):
        p = c % 2
        pltpu.sync_copy(x_hbm.at[pl.ds(base + c * CHT, CHT)], bufs[p])
        cp0 = pltpu.async_copy(bufs[p], xg_hbm.at[i0_v.at[c]], sems[p])
        cp1 = pltpu.async_copy(bufs[p], xg_hbm.at[i1_v.at[c]], sems[p])
        copies[p] = (cp0, cp1)

    def finish(c):
        p = c % 2
        copies[p][0].wait()
        copies[p][1].wait()

    start(0)
    for c in range(1, NCH):
        start(c)
        finish(c - 1)
    finish(NCH - 1)


def _mlp_kernel(src_ref, ebk_ref, nb_ref, x_any, w1_ref, w2_ref, out_ref,
                xb, sem0, sem1):
    b = pl.program_id(0)
    nb = nb_ref[0]
    par = b % 2

    def issue(bb, xslot, sem):
        def one(j, _):
            s = src_ref[bb * B + j]
            pltpu.make_async_copy(
                x_any.at[pl.ds(s, 1), :], xslot.at[pl.ds(j, 1), :],
                sem).start()
            return 0
        lax.fori_loop(0, B, one, 0, unroll=4)

    def drain_compute(bb, xslot, sem):
        def one(j, _):
            s = src_ref[bb * B + j]
            pltpu.make_async_copy(
                x_any.at[pl.ds(s, 1), :], xslot.at[pl.ds(j, 1), :],
                sem).wait()
            return 0
        lax.fori_loop(0, B, one, 0, unroll=4)
        x = xslot[...]
        h = jax.lax.dot_general(
            x, w1_ref[0], (((1,), (0,)), ((), ())),
            preferred_element_type=jnp.float32)
        h = h * jax.nn.sigmoid(h)
        y = jax.lax.dot_general(
            h, w2_ref[0], (((1,), (0,)), ((), ())),
            preferred_element_type=jnp.float32)
        out_ref[...] = y

    @pl.when(b == 0)
    def _():
        issue(0, xb.at[0], sem0)

    @pl.when((par == 0) & (b + 1 < nb))
    def _():
        issue(b + 1, xb.at[1], sem1)

    @pl.when((par == 1) & (b + 1 < nb))
    def _():
        issue(b + 1, xb.at[0], sem0)

    @pl.when((par == 0) & (b < nb))
    def _():
        drain_compute(b, xb.at[0], sem0)

    @pl.when((par == 1) & (b < nb))
    def _():
        drain_compute(b, xb.at[1], sem1)


def _sc_combine(yg_hbm, d0_hbm, d1_hbm, w0_hbm, w1_hbm, out_hbm,
                i0_v, i1_v, w0_v, w1_v, r0a, r1a, r0b, r1b, sem_a, sem_b):
    wid = lax.axis_index("s") * 2 + lax.axis_index("c")
    base = wid * TPW
    pltpu.sync_copy(d0_hbm.at[pl.ds(base, TPW)], i0_v)
    pltpu.sync_copy(d1_hbm.at[pl.ds(base, TPW)], i1_v)
    pltpu.sync_copy(w0_hbm.at[pl.ds(base, TPW)], w0_v)
    pltpu.sync_copy(w1_hbm.at[pl.ds(base, TPW)], w1_v)
    bufs = ((r0a, r1a), (r0b, r1b))
    sems = (sem_a, sem_b)
    copies = [None, None]

    def start(c):
        p = c % 2
        cp0 = pltpu.async_copy(
            yg_hbm.at[i0_v.at[pl.ds(c * CHC, CHC)]], bufs[p][0], sems[p])
        cp1 = pltpu.async_copy(
            yg_hbm.at[i1_v.at[pl.ds(c * CHC, CHC)]], bufs[p][1], sems[p])
        copies[p] = (cp0, cp1)

    def finish(c):
        p = c % 2
        copies[p][0].wait()
        copies[p][1].wait()
        r0, r1 = bufs[p]

        base16 = (c * CHC // L) * L
        w16_0 = w0_v[pl.ds(base16, L)]
        w16_1 = w1_v[pl.ds(base16, L)]

        gdn = lax.GatherDimensionNumbers(
            offset_dims=(), collapsed_slice_dims=(0,), start_index_map=(0,))

        def body(j, _):
            jloc = jnp.zeros((L,), jnp.int32) + (c * CHC - base16 + j)
            ws0 = lax.gather(w16_0, jloc[:, None], gdn, (1,),
                             mode=lax.GatherScatterMode.PROMISE_IN_BOUNDS)
            ws1 = lax.gather(w16_1, jloc[:, None], gdn, (1,),
                             mode=lax.GatherScatterMode.PROMISE_IN_BOUNDS)
            for k in range(D // L):
                a = r0[j, pl.ds(k * L, L)]
                bb = r1[j, pl.ds(k * L, L)]
                r0[j, pl.ds(k * L, L)] = a * ws0 + bb * ws1
            return 0

        lax.fori_loop(0, CHC, body, 0)
        pltpu.sync_copy(r0, out_hbm.at[pl.ds(base + c * CHC, CHC)])

    start(0)
    for c in range(1, NCC):
        start(c)
        finish(c - 1)
    finish(NCC - 1)


@jax.jit
def kernel(hidden_states, gate_w, experts_w1, experts_w2):
    f32 = jnp.float32
    x = hidden_states.astype(f32)
    gwt = jnp.zeros((D, EP), f32).at[:, :E].set(gate_w.T.astype(f32))

    router = pl.pallas_call(
        _router_kernel,
        grid=(T // BT,),
        in_specs=[
            pl.BlockSpec((BT, D), lambda b: (b, 0)),
            pl.BlockSpec((D, EP), lambda b: (0, 0)),
        ],
        out_specs=[
            pl.BlockSpec((BT, EP), lambda b: (b, 0)),
            pl.BlockSpec((BT, EP), lambda b: (b, 0)),
            pl.BlockSpec((BT, EP), lambda b: (b, 0)),
            pl.BlockSpec((BT, EP), lambda b: (b, 0)),
            pl.BlockSpec((BT, EP), lambda b: (b, 0)),
            pl.BlockSpec((BT, EP), lambda b: (b, 0)),
            pl.BlockSpec((8, EP), lambda b: (0, 0)),
        ],
        out_shape=[
            jax.ShapeDtypeStruct((T, EP), jnp.int32),
            jax.ShapeDtypeStruct((T, EP), jnp.int32),
            jax.ShapeDtypeStruct((T, EP), f32),
            jax.ShapeDtypeStruct((T, EP), f32),
            jax.ShapeDtypeStruct((T, EP), f32),
            jax.ShapeDtypeStruct((T, EP), f32),
            jax.ShapeDtypeStruct((8, EP), f32),
        ],
        scratch_shapes=[pltpu.VMEM((8, EP), f32)],
    )
    i0a, i1a, w0a, w1a, p0a, p1a, cnta = router(x, gwt)

    i0 = i0a[:, 0]
    i1 = i1a[:, 0]
    w0 = w0a[:, 0]
    w1 = w1a[:, 0]
    p0 = p0a[:, 0].astype(jnp.int32)
    p1 = p1a[:, 0].astype(jnp.int32)
    counts = cnta[0, :E].astype(jnp.int32)

    blocks_e = (counts + B - 1) // B
    blk_start = jnp.concatenate(
        [jnp.zeros((1,), jnp.int32), jnp.cumsum(blocks_e).astype(jnp.int32)])
    seg_off = blk_start * B
    nb_used = blk_start[E]
    barange = jnp.arange(NB, dtype=jnp.int32)
    e_blk = jnp.sum(
        (barange[:, None] >= blk_start[1:][None, :]).astype(jnp.int32), axis=1)
    e_last = jnp.minimum(e_blk[jnp.maximum(nb_used - 1, 0)], E - 1)
    e_blk = jnp.where(barange < nb_used, jnp.minimum(e_blk, E - 1), e_last)
    dest0 = seg_off[i0] + p0
    dest1 = seg_off[i1] + p1
    tok = jnp.arange(T, dtype=jnp.int32)
    src = jnp.zeros((NR,), jnp.int32).at[dest0].set(tok).at[dest1].set(tok)

    mlp = pl.pallas_call(
        _mlp_kernel,
        grid_spec=pltpu.PrefetchScalarGridSpec(
            num_scalar_prefetch=3,
            grid=(NB,),
            in_specs=[
                pl.BlockSpec(memory_space=pl.ANY),
                pl.BlockSpec((1, D, F), lambda b, src, ebk, nb: (ebk[b], 0, 0)),
                pl.BlockSpec((1, F, D), lambda b, src, ebk, nb: (ebk[b], 0, 0)),
            ],
            out_specs=pl.BlockSpec((B, D), lambda b, src, ebk, nb: (b, 0)),
            scratch_shapes=[
                pltpu.VMEM((2, B, D), f32),
                pltpu.SemaphoreType.DMA,
                pltpu.SemaphoreType.DMA,
            ],
        ),
        out_shape=jax.ShapeDtypeStruct((NR, D), f32),
    )
    yg = mlp(src, e_blk, nb_used[None], x,
             experts_w1.astype(f32), experts_w2.astype(f32))

    combine = functools.partial(
        pl.kernel,
        mesh=plsc.VectorSubcoreMesh(core_axis_name="c", subcore_axis_name="s"),
        out_type=jax.ShapeDtypeStruct((T, D), f32),
        scratch_types=[
            pltpu.VMEM((TPW,), jnp.int32),
            pltpu.VMEM((TPW,), jnp.int32),
            pltpu.VMEM((TPW,), f32),
            pltpu.VMEM((TPW,), f32),
            pltpu.VMEM((CHC, D), f32),
            pltpu.VMEM((CHC, D), f32),
            pltpu.VMEM((CHC, D), f32),
            pltpu.VMEM((CHC, D), f32),
            pltpu.SemaphoreType.DMA,
            pltpu.SemaphoreType.DMA,
        ],
    )(_sc_combine)
    out = combine(yg, dest0, dest1, w0, w1)
    return out


# speedup vs baseline: 1.0158x; 1.0158x over previous
import functools

import jax
import jax.numpy as jnp
from jax import lax
from jax.experimental import pallas as pl
from jax.experimental.pallas import tpu as pltpu
from jax.experimental.pallas import tpu_sc as plsc

T = 2048
D = 2048
E = 8
F = 1408
TOPK = 2

BT = 256
B = 128
NB = (T * TOPK) // B + E
NR = NB * B
EP = 128

NW = 32
L = 16
TPW = T // NW
CHT = 16
NCH = TPW // CHT
CHC = 8
NCC = TPW // CHC


def _router_kernel(x_ref, gwt_ref, i0_ref, i1_ref, w0_ref, w1_ref,
                   p0_ref, p1_ref, cnt_ref, cnt_scr):
    b = pl.program_id(0)

    @pl.when(b == 0)
    def _():
        cnt_scr[...] = jnp.zeros_like(cnt_scr)

    x = x_ref[...]
    logits = jax.lax.dot_general(
        x, gwt_ref[...], (((1,), (0,)), ((), ())),
        preferred_element_type=jnp.float32)
    col = jax.lax.broadcasted_iota(jnp.int32, logits.shape, 1)
    neg = jnp.float32(-1e30)
    logits = jnp.where(col < E, logits, neg)
    m0 = jnp.max(logits, axis=1, keepdims=True)
    i0 = jnp.min(jnp.where(logits == m0, col, EP), axis=1, keepdims=True)
    l2 = jnp.where(col == i0, neg, logits)
    m1 = jnp.max(l2, axis=1, keepdims=True)
    i1 = jnp.min(jnp.where(l2 == m1, col, EP), axis=1, keepdims=True)
    w0 = jax.nn.sigmoid(m0 - m1)
    w1 = 1.0 - w0

    oh0 = (col == i0).astype(jnp.float32)
    oh1 = (col == i1).astype(jnp.float32)
    occ = oh0 + oh1
    r = jax.lax.broadcasted_iota(jnp.int32, (BT, BT), 0)
    c = jax.lax.broadcasted_iota(jnp.int32, (BT, BT), 1)
    ls = (c < r).astype(jnp.float32)
    pref = jax.lax.dot_general(
        ls, occ, (((1,), (0,)), ((), ())),
        preferred_element_type=jnp.float32)
    cnt = cnt_scr[0:1, :]
    base = pref + cnt
    p0 = jnp.sum(oh0 * base, axis=1, keepdims=True)
    p1 = jnp.sum(oh1 * base, axis=1, keepdims=True)
    cnt_new = cnt + jnp.sum(occ, axis=0, keepdims=True)
    cnt_scr[...] = jnp.broadcast_to(cnt_new, cnt_scr.shape)
    cnt_ref[...] = jnp.broadcast_to(cnt_new, cnt_ref.shape)

    i0_ref[...] = jnp.broadcast_to(i0, i0_ref.shape)
    i1_ref[...] = jnp.broadcast_to(i1, i1_ref.shape)
    w0_ref[...] = jnp.broadcast_to(w0, w0_ref.shape)
    w1_ref[...] = jnp.broadcast_to(w1, w1_ref.shape)
    p0_ref[...] = jnp.broadcast_to(p0, p0_ref.shape)
    p1_ref[...] = jnp.broadcast_to(p1, p1_ref.shape)


def _sc_dispatch(x_hbm, d0_hbm, d1_hbm, xg_hbm, i0_v, i1_v,
                 rows_a, rows_b, sem_a, sem_b):
    wid = lax.axis_index("s") * 2 + lax.axis_index("c")
    base = wid * TPW
    pltpu.sync_copy(d0_hbm.at[pl.ds(wid * NCH, NCH)], i0_v)
    pltpu.sync_copy(d1_hbm.at[pl.ds(wid * NCH, NCH)], i1_v)
    bufs = (rows_a, rows_b)
    sems = (sem_a, sem_b)
    copies = [None, None]

    def start(c):
        p = c % 2
        pltpu.sync_copy(x_hbm.at[pl.ds(base + c * CHT, CHT)], bufs[p])
        cp0 = pltpu.async_copy(bufs[p], xg_hbm.at[i0_v.at[c]], sems[p])
        cp1 = pltpu.async_copy(bufs[p], xg_hbm.at[i1_v.at[c]], sems[p])
        copies[p] = (cp0, cp1)

    def finish(c):
        p = c % 2
        copies[p][0].wait()
        copies[p][1].wait()

    start(0)
    for c in range(1, NCH):
        start(c)
        finish(c - 1)
    finish(NCH - 1)


def _mlp_kernel(src_ref, ebk_ref, nb_ref, x_any, w1_ref, w2_ref, out_ref,
                xb, sem0, sem1):
    b = pl.program_id(0)
    nb = nb_ref[0]
    par = b % 2

    def issue(bb, xslot, sem):
        def one(j, _):
            s = src_ref[bb * B + j]
            pltpu.make_async_copy(
                x_any.at[pl.ds(s, 1), :], xslot.at[pl.ds(j, 1), :],
                sem).start()
            return 0
        lax.fori_loop(0, B, one, 0, unroll=4)

    def drain_compute(bb, xslot, sem):
        pltpu.make_async_copy(x_any.at[pl.ds(0, B), :], xslot, sem).wait()
        x = xslot[...]
        h = jax.lax.dot_general(
            x, w1_ref[0], (((1,), (0,)), ((), ())),
            preferred_element_type=jnp.float32)
        h = h * jax.nn.sigmoid(h)
        y = jax.lax.dot_general(
            h, w2_ref[0], (((1,), (0,)), ((), ())),
            preferred_element_type=jnp.float32)
        out_ref[...] = y

    @pl.when(b == 0)
    def _():
        issue(0, xb.at[0], sem0)

    @pl.when((par == 0) & (b + 1 < nb))
    def _():
        issue(b + 1, xb.at[1], sem1)

    @pl.when((par == 1) & (b + 1 < nb))
    def _():
        issue(b + 1, xb.at[0], sem0)

    @pl.when((par == 0) & (b < nb))
    def _():
        drain_compute(b, xb.at[0], sem0)

    @pl.when((par == 1) & (b < nb))
    def _():
        drain_compute(b, xb.at[1], sem1)


def _sc_combine(yg_hbm, d0_hbm, d1_hbm, w0_hbm, w1_hbm, out_hbm,
                i0_v, i1_v, w0_v, w1_v, r0a, r1a, r0b, r1b, sem_a, sem_b):
    wid = lax.axis_index("s") * 2 + lax.axis_index("c")
    base = wid * TPW
    pltpu.sync_copy(d0_hbm.at[pl.ds(base, TPW)], i0_v)
    pltpu.sync_copy(d1_hbm.at[pl.ds(base, TPW)], i1_v)
    pltpu.sync_copy(w0_hbm.at[pl.ds(base, TPW)], w0_v)
    pltpu.sync_copy(w1_hbm.at[pl.ds(base, TPW)], w1_v)
    bufs = ((r0a, r1a), (r0b, r1b))
    sems = (sem_a, sem_b)
    copies = [None, None]

    def start(c):
        p = c % 2
        cp0 = pltpu.async_copy(
            yg_hbm.at[i0_v.at[pl.ds(c * CHC, CHC)]], bufs[p][0], sems[p])
        cp1 = pltpu.async_copy(
            yg_hbm.at[i1_v.at[pl.ds(c * CHC, CHC)]], bufs[p][1], sems[p])
        copies[p] = (cp0, cp1)

    def finish(c):
        p = c % 2
        copies[p][0].wait()
        copies[p][1].wait()
        r0, r1 = bufs[p]

        base16 = (c * CHC // L) * L
        w16_0 = w0_v[pl.ds(base16, L)]
        w16_1 = w1_v[pl.ds(base16, L)]

        gdn = lax.GatherDimensionNumbers(
            offset_dims=(), collapsed_slice_dims=(0,), start_index_map=(0,))

        def body(j, _):
            jloc = jnp.zeros((L,), jnp.int32) + (c * CHC - base16 + j)
            ws0 = lax.gather(w16_0, jloc[:, None], gdn, (1,),
                             mode=lax.GatherScatterMode.PROMISE_IN_BOUNDS)
            ws1 = lax.gather(w16_1, jloc[:, None], gdn, (1,),
                             mode=lax.GatherScatterMode.PROMISE_IN_BOUNDS)
            for k in range(D // L):
                a = r0[j, pl.ds(k * L, L)]
                bb = r1[j, pl.ds(k * L, L)]
                r0[j, pl.ds(k * L, L)] = a * ws0 + bb * ws1
            return 0

        lax.fori_loop(0, CHC, body, 0)
        pltpu.sync_copy(r0, out_hbm.at[pl.ds(base + c * CHC, CHC)])

    start(0)
    for c in range(1, NCC):
        start(c)
        finish(c - 1)
    finish(NCC - 1)


@jax.jit
def kernel(hidden_states, gate_w, experts_w1, experts_w2):
    f32 = jnp.float32
    x = hidden_states.astype(f32)
    gwt = jnp.zeros((D, EP), f32).at[:, :E].set(gate_w.T.astype(f32))

    router = pl.pallas_call(
        _router_kernel,
        grid=(T // BT,),
        in_specs=[
            pl.BlockSpec((BT, D), lambda b: (b, 0)),
            pl.BlockSpec((D, EP), lambda b: (0, 0)),
        ],
        out_specs=[
            pl.BlockSpec((BT, EP), lambda b: (b, 0)),
            pl.BlockSpec((BT, EP), lambda b: (b, 0)),
            pl.BlockSpec((BT, EP), lambda b: (b, 0)),
            pl.BlockSpec((BT, EP), lambda b: (b, 0)),
            pl.BlockSpec((BT, EP), lambda b: (b, 0)),
            pl.BlockSpec((BT, EP), lambda b: (b, 0)),
            pl.BlockSpec((8, EP), lambda b: (0, 0)),
        ],
        out_shape=[
            jax.ShapeDtypeStruct((T, EP), jnp.int32),
            jax.ShapeDtypeStruct((T, EP), jnp.int32),
            jax.ShapeDtypeStruct((T, EP), f32),
            jax.ShapeDtypeStruct((T, EP), f32),
            jax.ShapeDtypeStruct((T, EP), f32),
            jax.ShapeDtypeStruct((T, EP), f32),
            jax.ShapeDtypeStruct((8, EP), f32),
        ],
        scratch_shapes=[pltpu.VMEM((8, EP), f32)],
    )
    i0a, i1a, w0a, w1a, p0a, p1a, cnta = router(x, gwt)

    i0 = i0a[:, 0]
    i1 = i1a[:, 0]
    w0 = w0a[:, 0]
    w1 = w1a[:, 0]
    p0 = p0a[:, 0].astype(jnp.int32)
    p1 = p1a[:, 0].astype(jnp.int32)
    counts = cnta[0, :E].astype(jnp.int32)

    blocks_e = (counts + B - 1) // B
    blk_start = jnp.concatenate(
        [jnp.zeros((1,), jnp.int32), jnp.cumsum(blocks_e).astype(jnp.int32)])
    seg_off = blk_start * B
    nb_used = blk_start[E]
    barange = jnp.arange(NB, dtype=jnp.int32)
    e_blk = jnp.sum(
        (barange[:, None] >= blk_start[1:][None, :]).astype(jnp.int32), axis=1)
    e_last = jnp.minimum(e_blk[jnp.maximum(nb_used - 1, 0)], E - 1)
    e_blk = jnp.where(barange < nb_used, jnp.minimum(e_blk, E - 1), e_last)
    dest0 = seg_off[i0] + p0
    dest1 = seg_off[i1] + p1
    tok = jnp.arange(T, dtype=jnp.int32)
    src = jnp.zeros((NR,), jnp.int32).at[dest0].set(tok).at[dest1].set(tok)

    mlp = pl.pallas_call(
        _mlp_kernel,
        grid_spec=pltpu.PrefetchScalarGridSpec(
            num_scalar_prefetch=3,
            grid=(NB,),
            in_specs=[
                pl.BlockSpec(memory_space=pl.ANY),
                pl.BlockSpec((1, D, F), lambda b, src, ebk, nb: (ebk[b], 0, 0)),
                pl.BlockSpec((1, F, D), lambda b, src, ebk, nb: (ebk[b], 0, 0)),
            ],
            out_specs=pl.BlockSpec((B, D), lambda b, src, ebk, nb: (b, 0)),
            scratch_shapes=[
                pltpu.VMEM((2, B, D), f32),
                pltpu.SemaphoreType.DMA,
                pltpu.SemaphoreType.DMA,
            ],
        ),
        out_shape=jax.ShapeDtypeStruct((NR, D), f32),
    )
    yg = mlp(src, e_blk, nb_used[None], x,
             experts_w1.astype(f32), experts_w2.astype(f32))

    combine = functools.partial(
        pl.kernel,
        mesh=plsc.VectorSubcoreMesh(core_axis_name="c", subcore_axis_name="s"),
        out_type=jax.ShapeDtypeStruct((T, D), f32),
        scratch_types=[
            pltpu.VMEM((TPW,), jnp.int32),
            pltpu.VMEM((TPW,), jnp.int32),
            pltpu.VMEM((TPW,), f32),
            pltpu.VMEM((TPW,), f32),
            pltpu.VMEM((CHC, D), f32),
            pltpu.VMEM((CHC, D), f32),
            pltpu.VMEM((CHC, D), f32),
            pltpu.VMEM((CHC, D), f32),
            pltpu.SemaphoreType.DMA,
            pltpu.SemaphoreType.DMA,
        ],
    )(_sc_combine)
    out = combine(yg, dest0, dest1, w0, w1)
    return out


# speedup vs baseline: 1.1381x; 1.1204x over previous
import functools

import jax
import jax.numpy as jnp
from jax import lax
from jax.experimental import pallas as pl
from jax.experimental.pallas import tpu as pltpu
from jax.experimental.pallas import tpu_sc as plsc

T = 2048
D = 2048
E = 8
F = 1408
TOPK = 2

BT = 256
B = 128
NB = (T * TOPK) // B + E
NR = NB * B
EP = 128

NW = 32
L = 16
TPW = T // NW
CHT = 16
NCH = TPW // CHT
CHC = 8
NCC = TPW // CHC


def _router_kernel(x_ref, gwt_ref, iv_ref, w0_ref, cnt_ref, cnt_scr):
    b = pl.program_id(0)

    @pl.when(b == 0)
    def _():
        cnt_scr[...] = jnp.zeros_like(cnt_scr)

    x = x_ref[...]
    logits = jax.lax.dot_general(
        x, gwt_ref[...], (((1,), (0,)), ((), ())),
        preferred_element_type=jnp.float32)
    col = jax.lax.broadcasted_iota(jnp.int32, logits.shape, 1)
    neg = jnp.float32(-1e30)
    logits = jnp.where(col < E, logits, neg)
    m0 = jnp.max(logits, axis=1, keepdims=True)
    i0 = jnp.min(jnp.where(logits == m0, col, EP), axis=1, keepdims=True)
    l2 = jnp.where(col == i0, neg, logits)
    m1 = jnp.max(l2, axis=1, keepdims=True)
    i1 = jnp.min(jnp.where(l2 == m1, col, EP), axis=1, keepdims=True)
    w0 = jax.nn.sigmoid(m0 - m1)
    w1 = 1.0 - w0

    oh0 = (col == i0).astype(jnp.float32)
    oh1 = (col == i1).astype(jnp.float32)
    occ = oh0 + oh1
    r = jax.lax.broadcasted_iota(jnp.int32, (BT, BT), 0)
    c = jax.lax.broadcasted_iota(jnp.int32, (BT, BT), 1)
    ls = (c < r).astype(jnp.float32)
    pref = jax.lax.dot_general(
        ls, occ, (((1,), (0,)), ((), ())),
        preferred_element_type=jnp.float32)
    cnt = cnt_scr[0:1, :]
    base = pref + cnt
    p0 = jnp.sum(oh0 * base, axis=1, keepdims=True)
    p1 = jnp.sum(oh1 * base, axis=1, keepdims=True)
    cnt_new = cnt + jnp.sum(occ, axis=0, keepdims=True)
    cnt_scr[...] = jnp.broadcast_to(cnt_new, cnt_scr.shape)
    cnt_ref[...] = jnp.broadcast_to(cnt_new, cnt_ref.shape)

    p0i = p0.astype(jnp.int32)
    p1i = p1.astype(jnp.int32)
    iv = jnp.where(col == 0, i0,
                   jnp.where(col == 1, i1,
                             jnp.where(col == 2, p0i, p1i)))
    iv_ref[...] = iv
    w0_ref[...] = jnp.broadcast_to(w0, w0_ref.shape)


def _sc_dispatch(x_hbm, d0_hbm, d1_hbm, xg_hbm, i0_v, i1_v,
                 rows_a, rows_b, sem_a, sem_b):
    wid = lax.axis_index("s") * 2 + lax.axis_index("c")
    base = wid * TPW
    pltpu.sync_copy(d0_hbm.at[pl.ds(wid * NCH, NCH)], i0_v)
    pltpu.sync_copy(d1_hbm.at[pl.ds(wid * NCH, NCH)], i1_v)
    bufs = (rows_a, rows_b)
    sems = (sem_a, sem_b)
    copies = [None, None]

    def start(c):
        p = c % 2
        pltpu.sync_copy(x_hbm.at[pl.ds(base + c * CHT, CHT)], bufs[p])
        cp0 = pltpu.async_copy(bufs[p], xg_hbm.at[i0_v.at[c]], sems[p])
        cp1 = pltpu.async_copy(bufs[p], xg_hbm.at[i1_v.at[c]], sems[p])
        copies[p] = (cp0, cp1)

    def finish(c):
        p = c % 2
        copies[p][0].wait()
        copies[p][1].wait()

    start(0)
    for c in range(1, NCH):
        start(c)
        finish(c - 1)
    finish(NCH - 1)


def _mlp_kernel(ebk_ref, nb_ref, xg_ref, w1_ref, w2_ref, out_ref):
    b = pl.program_id(0)

    @pl.when(b < nb_ref[0])
    def _():
        x = xg_ref[...]
        h = jax.lax.dot_general(
            x, w1_ref[0], (((1,), (0,)), ((), ())),
            preferred_element_type=jnp.float32)
        h = h * jax.nn.sigmoid(h)
        y = jax.lax.dot_general(
            h, w2_ref[0], (((1,), (0,)), ((), ())),
            preferred_element_type=jnp.float32)
        out_ref[...] = y


def _sc_combine(yg_hbm, d0_hbm, d1_hbm, w0_hbm, w1_hbm, out_hbm,
                i0_v, i1_v, w0_v, w1_v, r0a, r1a, r0b, r1b, sem_a, sem_b):
    wid = lax.axis_index("s") * 2 + lax.axis_index("c")
    base = wid * TPW
    pltpu.sync_copy(d0_hbm.at[pl.ds(base, TPW)], i0_v)
    pltpu.sync_copy(d1_hbm.at[pl.ds(base, TPW)], i1_v)
    pltpu.sync_copy(w0_hbm.at[pl.ds(base, TPW)], w0_v)
    pltpu.sync_copy(w1_hbm.at[pl.ds(base, TPW)], w1_v)
    bufs = ((r0a, r1a), (r0b, r1b))
    sems = (sem_a, sem_b)
    copies = [None, None]

    def start(c):
        p = c % 2
        cp0 = pltpu.async_copy(
            yg_hbm.at[i0_v.at[pl.ds(c * CHC, CHC)]], bufs[p][0], sems[p])
        cp1 = pltpu.async_copy(
            yg_hbm.at[i1_v.at[pl.ds(c * CHC, CHC)]], bufs[p][1], sems[p])
        copies[p] = (cp0, cp1)

    def finish(c):
        p = c % 2
        copies[p][0].wait()
        copies[p][1].wait()
        r0, r1 = bufs[p]

        base16 = (c * CHC // L) * L
        w16_0 = w0_v[pl.ds(base16, L)]
        w16_1 = w1_v[pl.ds(base16, L)]

        gdn = lax.GatherDimensionNumbers(
            offset_dims=(), collapsed_slice_dims=(0,), start_index_map=(0,))

        def body(j, _):
            jloc = jnp.zeros((L,), jnp.int32) + (c * CHC - base16 + j)
            ws0 = lax.gather(w16_0, jloc[:, None], gdn, (1,),
                             mode=lax.GatherScatterMode.PROMISE_IN_BOUNDS)
            ws1 = lax.gather(w16_1, jloc[:, None], gdn, (1,),
                             mode=lax.GatherScatterMode.PROMISE_IN_BOUNDS)
            for k in range(D // L):
                a = r0[j, pl.ds(k * L, L)]
                bb = r1[j, pl.ds(k * L, L)]
                r0[j, pl.ds(k * L, L)] = a * ws0 + bb * ws1
            return 0

        lax.fori_loop(0, CHC, body, 0)
        pltpu.sync_copy(r0, out_hbm.at[pl.ds(base + c * CHC, CHC)])

    start(0)
    for c in range(1, NCC):
        start(c)
        finish(c - 1)
    finish(NCC - 1)


@jax.jit
def kernel(hidden_states, gate_w, experts_w1, experts_w2):
    f32 = jnp.float32
    x = hidden_states.astype(f32)
    gwt = jnp.zeros((D, EP), f32).at[:, :E].set(gate_w.T.astype(f32))

    router = pl.pallas_call(
        _router_kernel,
        grid=(T // BT,),
        in_specs=[
            pl.BlockSpec((BT, D), lambda b: (b, 0)),
            pl.BlockSpec((D, EP), lambda b: (0, 0)),
        ],
        out_specs=[
            pl.BlockSpec((BT, EP), lambda b: (b, 0)),
            pl.BlockSpec((BT, EP), lambda b: (b, 0)),
            pl.BlockSpec((8, EP), lambda b: (0, 0)),
        ],
        out_shape=[
            jax.ShapeDtypeStruct((T, EP), jnp.int32),
            jax.ShapeDtypeStruct((T, EP), f32),
            jax.ShapeDtypeStruct((8, EP), f32),
        ],
        scratch_shapes=[pltpu.VMEM((8, EP), f32)],
    )
    iva, w0a, cnta = router(x, gwt)

    i0 = iva[:, 0]
    i1 = iva[:, 1]
    p0 = iva[:, 2]
    p1 = iva[:, 3]
    w0 = w0a[:, 0]
    w1 = 1.0 - w0
    counts = cnta[0, :E].astype(jnp.int32)

    blocks_e = (counts + B - 1) // B
    blk_start = jnp.concatenate(
        [jnp.zeros((1,), jnp.int32), jnp.cumsum(blocks_e).astype(jnp.int32)])
    seg_off = blk_start * B
    nb_used = blk_start[E]
    barange = jnp.arange(NB, dtype=jnp.int32)
    e_blk = jnp.sum(
        (barange[:, None] >= blk_start[1:][None, :]).astype(jnp.int32), axis=1)
    e_last = jnp.minimum(e_blk[jnp.maximum(nb_used - 1, 0)], E - 1)
    e_blk = jnp.where(barange < nb_used, jnp.minimum(e_blk, E - 1), e_last)
    dest0 = seg_off[i0] + p0
    dest1 = seg_off[i1] + p1

    dispatch = functools.partial(
        pl.kernel,
        mesh=plsc.VectorSubcoreMesh(core_axis_name="c", subcore_axis_name="s"),
        out_type=jax.ShapeDtypeStruct((NR, D), f32),
        scratch_types=[
            pltpu.VMEM((NCH, CHT), jnp.int32),
            pltpu.VMEM((NCH, CHT), jnp.int32),
            pltpu.VMEM((CHT, D), f32),
            pltpu.VMEM((CHT, D), f32),
            pltpu.SemaphoreType.DMA,
            pltpu.SemaphoreType.DMA,
        ],
    )(_sc_dispatch)
    xg = dispatch(x, dest0.reshape(T // CHT, CHT), dest1.reshape(T // CHT, CHT))

    mlp = pl.pallas_call(
        _mlp_kernel,
        grid_spec=pltpu.PrefetchScalarGridSpec(
            num_scalar_prefetch=2,
            grid=(NB,),
            in_specs=[
                pl.BlockSpec((B, D), lambda b, ebk, nb: (b, 0)),
                pl.BlockSpec((1, D, F), lambda b, ebk, nb: (ebk[b], 0, 0)),
                pl.BlockSpec((1, F, D), lambda b, ebk, nb: (ebk[b], 0, 0)),
            ],
            out_specs=pl.BlockSpec((B, D), lambda b, ebk, nb: (b, 0)),
        ),
        out_shape=jax.ShapeDtypeStruct((NR, D), f32),
    )
    yg = mlp(e_blk, nb_used[None], xg,
             experts_w1.astype(f32), experts_w2.astype(f32))

    combine = functools.partial(
        pl.kernel,
        mesh=plsc.VectorSubcoreMesh(core_axis_name="c", subcore_axis_name="s"),
        out_type=jax.ShapeDtypeStruct((T, D), f32),
        scratch_types=[
            pltpu.VMEM((TPW,), jnp.int32),
            pltpu.VMEM((TPW,), jnp.int32),
            pltpu.VMEM((TPW,), f32),
            pltpu.VMEM((TPW,), f32),
            pltpu.VMEM((CHC, D), f32),
            pltpu.VMEM((CHC, D), f32),
            pltpu.VMEM((CHC, D), f32),
            pltpu.VMEM((CHC, D), f32),
            pltpu.SemaphoreType.DMA,
            pltpu.SemaphoreType.DMA,
        ],
    )(_sc_combine)
    out = combine(yg, dest0, dest1, w0, w1)
    return out
